# Initial kernel scaffold; baseline (speedup 1.0000x reference)
import jax
import jax.numpy as jnp
from jax.experimental import pallas as pl


def kernel(x, edge_index, edge_attr, batch, conv0_eps, conv0_edge_w, conv0_edge_b, conv0_w1, conv0_b1, conv0_g1, conv0_be1, conv0_w2, conv0_b2, bn0_g, bn0_b, conv1_eps, conv1_edge_w, conv1_edge_b, conv1_w1, conv1_b1, conv1_g1, conv1_be1, conv1_w2, conv1_b2, cls_w1, cls_b1, cls_g1, cls_be1, cls_w2, cls_b2, cls_g2, cls_be2, cls_w3, cls_b3):
    raise NotImplementedError("write your pallas kernel here")


# speedup vs baseline: 2.8488x; 2.8488x over previous
import functools

import jax
import jax.numpy as jnp
from jax import lax
from jax.experimental import pallas as pl
from jax.experimental.pallas import tpu as pltpu
from jax.experimental.pallas import tpu_sc as plsc

N_NODES = 10000
N_EDGES = 320000
D_FEAT = 128
D_EDGE = 16
HID = 64
N_GRAPHS = 512
NEG = 0.01
BN_EPS = 1e-5

NC = 2
NS = 16
NW = NC * NS
EPW = N_EDGES // NW
CHUNK = 80
NCHUNK = EPW // CHUNK
NROWCH = N_NODES // CHUNK


def _lrelu(v):
    return jnp.where(v >= 0, v, NEG * v)


def _bn(v, g, b):
    m = jnp.mean(v, axis=0, keepdims=True)
    var = jnp.mean((v - m) ** 2, axis=0, keepdims=True)
    return (v - m) * lax.rsqrt(var + BN_EPS) * g + b


def _make_edge_pass(d):
    mesh = plsc.VectorSubcoreMesh(core_axis_name="c", subcore_axis_name="s")

    @functools.partial(
        pl.kernel,
        out_type=jax.ShapeDtypeStruct((NC, N_NODES, d), jnp.float32),
        mesh=mesh,
        scratch_types=[
            pltpu.VMEM((CHUNK,), jnp.int32),
            pltpu.VMEM((CHUNK,), jnp.int32),
            pltpu.VMEM((CHUNK, d), jnp.float32),
            pltpu.VMEM((CHUNK, d), jnp.float32),
            pltpu.VMEM_SHARED((N_NODES, d), jnp.float32),
            pltpu.SemaphoreType.DMA,
        ],
    )
    def edge_pass(x_hbm, src_hbm, dst_hbm, e_hbm, out_hbm,
                  src_v, dst_v, xbuf, ebuf, accum, gsem):
        cid = lax.axis_index("c")
        sid = lax.axis_index("s")
        wid = sid * NC + cid

        zv = jnp.zeros((16,), jnp.float32)

        def zrow(i, carry):
            for k in range(d // 16):
                xbuf[i, pl.ds(k * 16, 16)] = zv
            return carry

        lax.fori_loop(0, CHUNK, zrow, 0)
        for r in range((NROWCH + NS - 1) // NS):
            j = sid + r * NS

            @pl.when(j < NROWCH)
            def _():
                pltpu.sync_copy(xbuf, accum.at[pl.ds(j * CHUNK, CHUNK)])

        plsc.subcore_barrier()

        def chunk(c, carry):
            base = wid * EPW + c * CHUNK
            pltpu.sync_copy(src_hbm.at[pl.ds(base, CHUNK)], src_v)
            g = pltpu.async_copy(x_hbm.at[src_v], xbuf, gsem)
            pltpu.sync_copy(dst_hbm.at[pl.ds(base, CHUNK)], dst_v)
            pltpu.sync_copy(e_hbm.at[pl.ds(base, CHUNK)], ebuf)
            g.wait()

            def row(i, rc):
                for k in range(d // 16):
                    s = pl.ds(k * 16, 16)
                    ebuf[i, s] = jnp.maximum(xbuf[i, s] + ebuf[i, s], 0.0)
                return rc

            lax.fori_loop(0, CHUNK, row, 0)
            pltpu.sync_copy(ebuf, accum.at[dst_v], add=True)
            return carry

        lax.fori_loop(0, NCHUNK, chunk, 0)
        plsc.subcore_barrier()
        for r in range((NROWCH + NS - 1) // NS):
            j = sid + r * NS

            @pl.when(j < NROWCH)
            def _():
                pltpu.sync_copy(accum.at[pl.ds(j * CHUNK, CHUNK)],
                                out_hbm.at[cid, pl.ds(j * CHUNK, CHUNK)])

    return edge_pass


_edge_pass_128 = _make_edge_pass(D_FEAT)


EB = 3200


def _eproj_body(ea_ref, w0_ref, b0_ref, w1_ref, b1_ref, e0_ref, e1_ref):
    ea = ea_ref[...]
    e0_ref[...] = (jnp.dot(ea, w0_ref[...], preferred_element_type=jnp.float32)
                   + b0_ref[...])
    e1 = (jnp.dot(ea, w1_ref[...], preferred_element_type=jnp.float32)
          + b1_ref[...])
    e1_ref[...] = jnp.concatenate(
        [e1, jnp.zeros((EB, D_FEAT - HID), jnp.float32)], axis=1)


def _edge_proj(edge_attr, w0t, b0, w1t, b1):
    grid = (N_EDGES // EB,)
    return pl.pallas_call(
        _eproj_body,
        grid=grid,
        in_specs=[
            pl.BlockSpec((EB, D_EDGE), lambda i: (i, 0)),
            pl.BlockSpec((D_EDGE, D_FEAT), lambda i: (0, 0)),
            pl.BlockSpec((1, D_FEAT), lambda i: (0, 0)),
            pl.BlockSpec((D_EDGE, HID), lambda i: (0, 0)),
            pl.BlockSpec((1, HID), lambda i: (0, 0)),
        ],
        out_specs=[
            pl.BlockSpec((EB, D_FEAT), lambda i: (i, 0)),
            pl.BlockSpec((EB, D_FEAT), lambda i: (i, 0)),
        ],
        out_shape=[
            jax.ShapeDtypeStruct((N_EDGES, D_FEAT), jnp.float32),
            jax.ShapeDtypeStruct((N_EDGES, D_FEAT), jnp.float32),
        ],
    )(edge_attr, w0t, b0, w1t, b1)


def _node_body(final_bn, din, pad_out, x_ref, agg_ref, eps_ref, w1_ref,
               b1_ref, g1_ref, be1_ref, w2_ref, b2_ref, g_ref, b_ref,
               out_ref):
    h = (1.0 + eps_ref[...]) * x_ref[...] + agg_ref[0] + agg_ref[1]
    h = h[:, :din]
    h = jnp.dot(h, w1_ref[...], preferred_element_type=jnp.float32) + b1_ref[...]
    h = _lrelu(_bn(h, g1_ref[...], be1_ref[...]))
    h = jnp.dot(h, w2_ref[...], preferred_element_type=jnp.float32) + b2_ref[...]
    if final_bn:
        h = _bn(h, g_ref[...], b_ref[...])
    h = _lrelu(h)
    if pad_out:
        h = jnp.concatenate(
            [h, jnp.zeros((N_NODES, D_FEAT - HID), jnp.float32)], axis=1)
    out_ref[...] = h


def _node_update(final_bn, din, pad_out, x, agg, eps, w1t, b1, g1, be1, w2t,
                 b2, g, b):
    dout = D_FEAT if pad_out else HID
    full = lambda s: pl.BlockSpec(s, lambda: tuple(0 for _ in s))
    return pl.pallas_call(
        functools.partial(_node_body, final_bn, din, pad_out),
        in_specs=[
            full((N_NODES, x.shape[1])),
            full((NC, N_NODES, x.shape[1])),
            full((1, 1)),
            full((din, HID)),
            full((1, HID)),
            full((1, HID)),
            full((1, HID)),
            full((HID, HID)),
            full((1, HID)),
            full((1, HID)),
            full((1, HID)),
        ],
        out_specs=full((N_NODES, dout)),
        out_shape=jax.ShapeDtypeStruct((N_NODES, dout), jnp.float32),
    )(x, agg, eps, w1t, b1, g1, be1, w2t, b2, g, b)


CH = 1000


def _cls_body(h_ref, bcol_ref, wa_ref, wb_ref, b1_ref, g1_ref,
              be1_ref, w2_ref, b2_ref, g2_ref, be2_ref, w3_ref, b3_ref,
              out_ref, z_ref):
    grow = lax.broadcasted_iota(jnp.int32, (1, N_GRAPHS), 1)

    def pc(c, pool):
        hc = h_ref[pl.ds(c * CH, CH), :]
        bcol = bcol_ref[pl.ds(c * CH, CH), :]
        oh = (bcol == grow).astype(jnp.float32)
        return pool + lax.dot_general(
            oh, hc, (((0,), (0,)), ((), ())),
            preferred_element_type=jnp.float32,
            precision=lax.Precision.HIGHEST)

    pool = lax.fori_loop(0, N_NODES // CH, pc,
                         jnp.zeros((N_GRAPHS, HID), jnp.float32))

    def zc(c, carry):
        hc = h_ref[pl.ds(c * CH, CH), :]
        bcol = bcol_ref[pl.ds(c * CH, CH), :]
        oh = (bcol == grow).astype(jnp.float32)
        ctx = jnp.dot(oh, pool, preferred_element_type=jnp.float32,
                      precision=lax.Precision.HIGHEST)
        z_ref[pl.ds(c * CH, CH), :] = (
            jnp.dot(hc, wa_ref[...], preferred_element_type=jnp.float32)
            + jnp.dot(ctx, wb_ref[...], preferred_element_type=jnp.float32)
            + b1_ref[...])
        return carry

    lax.fori_loop(0, N_NODES // CH, zc, 0)

    z = _lrelu(_bn(z_ref[...], g1_ref[...], be1_ref[...]))
    z = jnp.dot(z, w2_ref[...], preferred_element_type=jnp.float32) + b2_ref[...]
    z = _lrelu(_bn(z, g2_ref[...], be2_ref[...]))
    out_ref[...] = (jnp.dot(z, w3_ref[...], preferred_element_type=jnp.float32)
                    + b3_ref[...])


def _cls_head(h, bcol, wat, wbt, b1, g1, be1, w2t, b2, g2, be2, w3t, b3):
    full = lambda s: pl.BlockSpec(s, lambda: tuple(0 for _ in s))
    return pl.pallas_call(
        _cls_body,
        in_specs=[
            full((N_NODES, HID)),
            full((N_NODES, 1)),
            full((HID, HID)),
            full((HID, HID)),
            full((1, HID)),
            full((1, HID)),
            full((1, HID)),
            full((HID, HID)),
            full((1, HID)),
            full((1, HID)),
            full((1, HID)),
            full((HID, 1)),
            full((1, 1)),
        ],
        out_specs=full((N_NODES, 1)),
        out_shape=jax.ShapeDtypeStruct((N_NODES, 1), jnp.float32),
        scratch_shapes=[pltpu.VMEM((N_NODES, HID), jnp.float32)],
    )(h, bcol, wat, wbt, b1, g1, be1, w2t, b2, g2, be2, w3t, b3)


def kernel(x, edge_index, edge_attr, batch, conv0_eps, conv0_edge_w,
           conv0_edge_b, conv0_w1, conv0_b1, conv0_g1, conv0_be1, conv0_w2,
           conv0_b2, bn0_g, bn0_b, conv1_eps, conv1_edge_w, conv1_edge_b,
           conv1_w1, conv1_b1, conv1_g1, conv1_be1, conv1_w2, conv1_b2,
           cls_w1, cls_b1, cls_g1, cls_be1, cls_w2, cls_b2, cls_g2, cls_be2,
           cls_w3, cls_b3):
    src = edge_index[0]
    dst = edge_index[1]
    row = lambda v: v.reshape(1, -1)

    e0, e1 = _edge_proj(edge_attr, conv0_edge_w.T, row(conv0_edge_b),
                        conv1_edge_w.T, row(conv1_edge_b))

    agg0 = _edge_pass_128(x, src, dst, e0)
    h1 = _node_update(True, D_FEAT, True, x, agg0, conv0_eps.reshape(1, 1),
                      conv0_w1.T, row(conv0_b1), row(conv0_g1),
                      row(conv0_be1), conv0_w2.T, row(conv0_b2), row(bn0_g),
                      row(bn0_b))

    agg1 = _edge_pass_128(h1, src, dst, e1)
    h2 = _node_update(False, HID, False, h1, agg1, conv1_eps.reshape(1, 1),
                      conv1_w1.T, row(conv1_b1), row(conv1_g1),
                      row(conv1_be1), conv1_w2.T, row(conv1_b2),
                      row(conv1_g1), row(conv1_be1))

    out = _cls_head(h2, batch.reshape(-1, 1),
                    cls_w1[:, :HID].T, cls_w1[:, HID:].T, row(cls_b1),
                    row(cls_g1), row(cls_be1), cls_w2.T, row(cls_b2),
                    row(cls_g2), row(cls_be2), cls_w3.T, cls_b3.reshape(1, 1))
    return out.reshape(-1)


# speedup vs baseline: 4.3166x; 1.5152x over previous
import functools

import jax
import jax.numpy as jnp
from jax import lax
from jax.experimental import pallas as pl
from jax.experimental.pallas import tpu as pltpu
from jax.experimental.pallas import tpu_sc as plsc

N_NODES = 10000
N_EDGES = 320000
D_FEAT = 128
D_EDGE = 16
HID = 64
N_GRAPHS = 512
NEG = 0.01
BN_EPS = 1e-5

NC = 2
NS = 16
NW = NC * NS
EPW = N_EDGES // NW
CHUNK = 40
NCHUNK = EPW // CHUNK
GRP = 10
TMAX = NCHUNK // GRP
WBCH = CHUNK
NROWCH = N_NODES // WBCH


def _lrelu(v):
    return jnp.where(v >= 0, v, NEG * v)


def _bn(v, g, b):
    m = jnp.mean(v, axis=0, keepdims=True)
    var = jnp.mean((v - m) ** 2, axis=0, keepdims=True)
    return (v - m) * lax.rsqrt(var + BN_EPS) * g + b


def _make_edge_pass(d):
    mesh = plsc.VectorSubcoreMesh(core_axis_name="c", subcore_axis_name="s")

    @functools.partial(
        pl.kernel,
        out_type=jax.ShapeDtypeStruct((NC, N_NODES, d), jnp.float32),
        mesh=mesh,
        scratch_types=[
            [pltpu.VMEM((CHUNK,), jnp.int32) for _ in range(GRP)],
            [pltpu.VMEM((CHUNK,), jnp.int32) for _ in range(GRP)],
            [pltpu.VMEM((CHUNK, d), jnp.float32) for _ in range(2)],
            [pltpu.VMEM((CHUNK, d), jnp.float32) for _ in range(2)],
            [pltpu.VMEM((CHUNK, d), jnp.float32) for _ in range(2)],
            pltpu.VMEM_SHARED((N_NODES, d), jnp.float32),
            [pltpu.SemaphoreType.DMA for _ in range(2)],
            [pltpu.SemaphoreType.DMA for _ in range(2)],
            [pltpu.SemaphoreType.DMA for _ in range(2)],
            [pltpu.SemaphoreType.DMA for _ in range(GRP)],
        ],
    )
    def edge_pass(x_hbm, src_hbm, dst_hbm, e_hbm, out_hbm,
                  sidx, didx, xbuf, ebuf, mbuf, accum,
                  gsem, esem, ssem, isem):
        cid = lax.axis_index("c")
        sid = lax.axis_index("s")
        wid = sid * NC + cid

        def _iload_s(c, q):
            return pltpu.make_async_copy(
                src_hbm.at[pl.ds(wid * EPW + c * CHUNK, CHUNK)], sidx[q],
                isem[q])

        def _iload_d(c, q):
            return pltpu.make_async_copy(
                dst_hbm.at[pl.ds(wid * EPW + c * CHUNK, CHUNK)], didx[q],
                isem[q])

        def _gather(c, b, q):
            return pltpu.make_async_copy(x_hbm.at[sidx[q]], xbuf[b], gsem[b])

        def _eload(c, b):
            return pltpu.make_async_copy(e_hbm.at[pl.ds(wid * EPW + c * CHUNK,
                                                        CHUNK)],
                                         ebuf[b], esem[b])

        def _scatter(b, q):
            return pltpu.make_async_copy(mbuf[b], accum.at[didx[q]], ssem[b])

        zv = jnp.zeros((16,), jnp.float32)

        def zrow(i, carry):
            for k in range(d // 16):
                mbuf[0][i, pl.ds(k * 16, 16)] = zv
            return carry

        lax.fori_loop(0, WBCH, zrow, 0)
        for r in range((NROWCH + NS - 1) // NS):
            j = sid + r * NS

            @pl.when(j < NROWCH)
            def _():
                pltpu.sync_copy(mbuf[0], accum.at[pl.ds(j * WBCH, WBCH)])

        plsc.subcore_barrier()

        for c0 in range(4):
            _iload_s(c0, c0).start()
            _iload_d(c0, c0).start()
        for b in range(2):
            _iload_s(b, b).wait()
            _iload_d(b, b).wait()
            _gather(b, b, b).start()
            _eload(b, b).start()

        def group(t, carry):
            for u in range(GRP):
                c = GRP * t + u
                b = u % 2
                _gather(c, b, u).wait()
                _eload(c, b).wait()

                if u >= 2:
                    _scatter(b, (u - 2) % GRP).wait()
                else:
                    @pl.when(t >= 1)
                    def _():
                        _scatter(b, (u - 2) % GRP).wait()

                if u < 6:
                    _iload_s(c + 4, (u + 4) % GRP).start()
                    _iload_d(c + 4, (u + 4) % GRP).start()
                else:
                    @pl.when(t + 1 < TMAX)
                    def _():
                        _iload_s(c + 4, (u + 4) % GRP).start()
                        _iload_d(c + 4, (u + 4) % GRP).start()

                def row(i, rc):
                    for k in range(d // 16):
                        s = pl.ds(k * 16, 16)
                        mbuf[b][i, s] = jnp.maximum(
                            xbuf[b][i, s] + ebuf[b][i, s], 0.0)
                    return rc

                lax.fori_loop(0, CHUNK, row, 0)
                _scatter(b, u).start(add=True)

                if u < 8:
                    _iload_s(c + 2, (u + 2) % GRP).wait()
                    _iload_d(c + 2, (u + 2) % GRP).wait()
                    _gather(c + 2, b, (u + 2) % GRP).start()
                    _eload(c + 2, b).start()
                else:
                    @pl.when(t + 1 < TMAX)
                    def _():
                        _iload_s(c + 2, (u + 2) % GRP).wait()
                        _iload_d(c + 2, (u + 2) % GRP).wait()
                        _gather(c + 2, b, (u + 2) % GRP).start()
                        _eload(c + 2, b).start()
            return carry

        lax.fori_loop(0, TMAX, group, 0)
        for b in range(2):
            _scatter(b, (NCHUNK - 2 + b) % GRP).wait()

        plsc.subcore_barrier()
        for r in range((NROWCH + NS - 1) // NS):
            j = sid + r * NS

            @pl.when(j < NROWCH)
            def _():
                pltpu.sync_copy(accum.at[pl.ds(j * WBCH, WBCH)],
                                out_hbm.at[cid, pl.ds(j * WBCH, WBCH)])

    return edge_pass


_edge_pass_128 = _make_edge_pass(D_FEAT)


EB = 3200


def _eproj_body(ea_ref, w0_ref, b0_ref, w1_ref, b1_ref, e0_ref, e1_ref):
    ea = ea_ref[...]
    e0_ref[...] = (jnp.dot(ea, w0_ref[...], preferred_element_type=jnp.float32)
                   + b0_ref[...])
    e1 = (jnp.dot(ea, w1_ref[...], preferred_element_type=jnp.float32)
          + b1_ref[...])
    e1_ref[...] = jnp.concatenate(
        [e1, jnp.zeros((EB, D_FEAT - HID), jnp.float32)], axis=1)


def _edge_proj(edge_attr, w0t, b0, w1t, b1):
    grid = (N_EDGES // EB,)
    return pl.pallas_call(
        _eproj_body,
        grid=grid,
        in_specs=[
            pl.BlockSpec((EB, D_EDGE), lambda i: (i, 0)),
            pl.BlockSpec((D_EDGE, D_FEAT), lambda i: (0, 0)),
            pl.BlockSpec((1, D_FEAT), lambda i: (0, 0)),
            pl.BlockSpec((D_EDGE, HID), lambda i: (0, 0)),
            pl.BlockSpec((1, HID), lambda i: (0, 0)),
        ],
        out_specs=[
            pl.BlockSpec((EB, D_FEAT), lambda i: (i, 0)),
            pl.BlockSpec((EB, D_FEAT), lambda i: (i, 0)),
        ],
        out_shape=[
            jax.ShapeDtypeStruct((N_EDGES, D_FEAT), jnp.float32),
            jax.ShapeDtypeStruct((N_EDGES, D_FEAT), jnp.float32),
        ],
    )(edge_attr, w0t, b0, w1t, b1)


def _node_body(final_bn, din, pad_out, x_ref, agg_ref, eps_ref, w1_ref,
               b1_ref, g1_ref, be1_ref, w2_ref, b2_ref, g_ref, b_ref,
               out_ref):
    h = (1.0 + eps_ref[...]) * x_ref[...] + agg_ref[0] + agg_ref[1]
    h = h[:, :din]
    h = jnp.dot(h, w1_ref[...], preferred_element_type=jnp.float32) + b1_ref[...]
    h = _lrelu(_bn(h, g1_ref[...], be1_ref[...]))
    h = jnp.dot(h, w2_ref[...], preferred_element_type=jnp.float32) + b2_ref[...]
    if final_bn:
        h = _bn(h, g_ref[...], b_ref[...])
    h = _lrelu(h)
    if pad_out:
        h = jnp.concatenate(
            [h, jnp.zeros((N_NODES, D_FEAT - HID), jnp.float32)], axis=1)
    out_ref[...] = h


def _node_update(final_bn, din, pad_out, x, agg, eps, w1t, b1, g1, be1, w2t,
                 b2, g, b):
    dout = D_FEAT if pad_out else HID
    full = lambda s: pl.BlockSpec(s, lambda: tuple(0 for _ in s))
    return pl.pallas_call(
        functools.partial(_node_body, final_bn, din, pad_out),
        in_specs=[
            full((N_NODES, x.shape[1])),
            full((NC, N_NODES, x.shape[1])),
            full((1, 1)),
            full((din, HID)),
            full((1, HID)),
            full((1, HID)),
            full((1, HID)),
            full((HID, HID)),
            full((1, HID)),
            full((1, HID)),
            full((1, HID)),
        ],
        out_specs=full((N_NODES, dout)),
        out_shape=jax.ShapeDtypeStruct((N_NODES, dout), jnp.float32),
    )(x, agg, eps, w1t, b1, g1, be1, w2t, b2, g, b)


CH = 1000


def _cls_body(h_ref, bcol_ref, wa_ref, wb_ref, b1_ref, g1_ref,
              be1_ref, w2_ref, b2_ref, g2_ref, be2_ref, w3_ref, b3_ref,
              out_ref, z_ref):
    grow = lax.broadcasted_iota(jnp.int32, (1, N_GRAPHS), 1)

    def pc(c, pool):
        hc = h_ref[pl.ds(c * CH, CH), :]
        bcol = bcol_ref[pl.ds(c * CH, CH), :]
        oh = (bcol == grow).astype(jnp.float32)
        return pool + lax.dot_general(
            oh, hc, (((0,), (0,)), ((), ())),
            preferred_element_type=jnp.float32,
            precision=lax.Precision.HIGHEST)

    pool = lax.fori_loop(0, N_NODES // CH, pc,
                         jnp.zeros((N_GRAPHS, HID), jnp.float32))

    def zc(c, carry):
        hc = h_ref[pl.ds(c * CH, CH), :]
        bcol = bcol_ref[pl.ds(c * CH, CH), :]
        oh = (bcol == grow).astype(jnp.float32)
        ctx = jnp.dot(oh, pool, preferred_element_type=jnp.float32,
                      precision=lax.Precision.HIGHEST)
        z_ref[pl.ds(c * CH, CH), :] = (
            jnp.dot(hc, wa_ref[...], preferred_element_type=jnp.float32)
            + jnp.dot(ctx, wb_ref[...], preferred_element_type=jnp.float32)
            + b1_ref[...])
        return carry

    lax.fori_loop(0, N_NODES // CH, zc, 0)

    z = _lrelu(_bn(z_ref[...], g1_ref[...], be1_ref[...]))
    z = jnp.dot(z, w2_ref[...], preferred_element_type=jnp.float32) + b2_ref[...]
    z = _lrelu(_bn(z, g2_ref[...], be2_ref[...]))
    out_ref[...] = (jnp.dot(z, w3_ref[...], preferred_element_type=jnp.float32)
                    + b3_ref[...])


def _cls_head(h, bcol, wat, wbt, b1, g1, be1, w2t, b2, g2, be2, w3t, b3):
    full = lambda s: pl.BlockSpec(s, lambda: tuple(0 for _ in s))
    return pl.pallas_call(
        _cls_body,
        in_specs=[
            full((N_NODES, HID)),
            full((N_NODES, 1)),
            full((HID, HID)),
            full((HID, HID)),
            full((1, HID)),
            full((1, HID)),
            full((1, HID)),
            full((HID, HID)),
            full((1, HID)),
            full((1, HID)),
            full((1, HID)),
            full((HID, 1)),
            full((1, 1)),
        ],
        out_specs=full((N_NODES, 1)),
        out_shape=jax.ShapeDtypeStruct((N_NODES, 1), jnp.float32),
        scratch_shapes=[pltpu.VMEM((N_NODES, HID), jnp.float32)],
    )(h, bcol, wat, wbt, b1, g1, be1, w2t, b2, g2, be2, w3t, b3)


def kernel(x, edge_index, edge_attr, batch, conv0_eps, conv0_edge_w,
           conv0_edge_b, conv0_w1, conv0_b1, conv0_g1, conv0_be1, conv0_w2,
           conv0_b2, bn0_g, bn0_b, conv1_eps, conv1_edge_w, conv1_edge_b,
           conv1_w1, conv1_b1, conv1_g1, conv1_be1, conv1_w2, conv1_b2,
           cls_w1, cls_b1, cls_g1, cls_be1, cls_w2, cls_b2, cls_g2, cls_be2,
           cls_w3, cls_b3):
    src = edge_index[0]
    dst = edge_index[1]
    row = lambda v: v.reshape(1, -1)

    e0, e1 = _edge_proj(edge_attr, conv0_edge_w.T, row(conv0_edge_b),
                        conv1_edge_w.T, row(conv1_edge_b))

    agg0 = _edge_pass_128(x, src, dst, e0)
    h1 = _node_update(True, D_FEAT, True, x, agg0, conv0_eps.reshape(1, 1),
                      conv0_w1.T, row(conv0_b1), row(conv0_g1),
                      row(conv0_be1), conv0_w2.T, row(conv0_b2), row(bn0_g),
                      row(bn0_b))

    agg1 = _edge_pass_128(h1, src, dst, e1)
    h2 = _node_update(False, HID, False, h1, agg1, conv1_eps.reshape(1, 1),
                      conv1_w1.T, row(conv1_b1), row(conv1_g1),
                      row(conv1_be1), conv1_w2.T, row(conv1_b2),
                      row(conv1_g1), row(conv1_be1))

    out = _cls_head(h2, batch.reshape(-1, 1),
                    cls_w1[:, :HID].T, cls_w1[:, HID:].T, row(cls_b1),
                    row(cls_g1), row(cls_be1), cls_w2.T, row(cls_b2),
                    row(cls_g2), row(cls_be2), cls_w3.T, cls_b3.reshape(1, 1))
    return out.reshape(-1)
